# baseline (speedup 1.0000x reference)
import jax
import jax.numpy as jnp
from jax import lax
from jax.experimental import pallas as pl
from jax.experimental.pallas import tpu as pltpu
from jax.experimental.pallas import tpu_sc as plsc

N = 1048576
GRID = 64
C = 64
NUM_CORES = 2
NUM_SUBCORES = 16
NTILES = NUM_CORES * NUM_SUBCORES
CH_GROUPS = 4
W_PER_TILE = 16
PT_GROUPS = NTILES // CH_GROUPS
PTS_PER_TILE = N // PT_GROUPS
P = 256
CHUNKS = PTS_PER_TILE // P
LANES = 16
ROWS_PER_DMA = 128

MASK_HI = -65536


def _halves_f32(v):
    lo = lax.bitcast_convert_type(lax.shift_left(v, 16), jnp.float32)
    hi = lax.bitcast_convert_type(v & MASK_HI, jnp.float32)
    return lo, hi


def _sc_body(x_hbm, y_hbm, tbl_hbm, g_hbm, b_hbm,
             tbl_v, x_v, y_v, out_v, sem):
    wid = lax.axis_index("s") * NUM_CORES + lax.axis_index("c")
    cb = wid & (CH_GROUPS - 1)
    pg = wid >> 2
    tile_base = pg * PTS_PER_TILE
    half = cb & 1

    pltpu.sync_copy(tbl_hbm.at[cb], tbl_v)

    lane_iota = lax.iota(jnp.int32, LANES)

    def chunk_body(ci, _):
        base = tile_base + ci * P
        pltpu.sync_copy(x_hbm.at[pl.ds(base, P)], x_v)
        pltpu.sync_copy(y_hbm.at[pl.ds(base, P)], y_v)

        def group_body(j, _):
            sl = pl.ds(j * LANES, LANES)
            xv = x_v[sl]
            yv = y_v[sl]
            ix = jnp.clip((xv + 1.0) * 0.5 * (GRID - 1), 0.0, float(GRID - 1))
            iy = jnp.clip((yv + 1.0) * 0.5 * (GRID - 1), 0.0, float(GRID - 1))
            x0 = jnp.minimum(ix.astype(jnp.int32), GRID - 2)
            y0 = jnp.minimum(iy.astype(jnp.int32), GRID - 2)
            wxf = ix - x0.astype(jnp.float32)
            wyf = iy - y0.astype(jnp.float32)
            w11 = wxf * wyf
            w01 = wxf - w11
            w10 = wyf - w11
            w00 = (1.0 - wxf) - w10
            r00 = (y0 * GRID + x0) * W_PER_TILE
            r01 = r00 + W_PER_TILE
            r10 = r00 + GRID * W_PER_TILE
            r11 = r10 + W_PER_TILE
            pvec = lane_iota + j * LANES
            for w in range(W_PER_TILE):
                e00, o00 = _halves_f32(plsc.load_gather(tbl_v, [r00 + w]))
                e01, o01 = _halves_f32(plsc.load_gather(tbl_v, [r01 + w]))
                e10, o10 = _halves_f32(plsc.load_gather(tbl_v, [r10 + w]))
                e11, o11 = _halves_f32(plsc.load_gather(tbl_v, [r11 + w]))
                even = w00 * e00 + w01 * e01 + w10 * e10 + w11 * e11
                odd = w00 * o00 + w01 * o01 + w10 * o10 + w11 * o11
                cidx = jnp.full((LANES,), 2 * w, jnp.int32)
                plsc.store_scatter(out_v, [pvec, cidx], even)
                plsc.store_scatter(out_v, [pvec, cidx + 1], odd)
            return _

        lax.fori_loop(0, P // LANES, group_body, None)

        @pl.when(cb < 2)
        def _w_g():
            pltpu.sync_copy(out_v, g_hbm.at[half, pl.ds(base, P)])

        @pl.when(cb >= 2)
        def _w_b():
            pltpu.sync_copy(out_v, b_hbm.at[half, pl.ds(base, P)])

        return _

    lax.fori_loop(0, CHUNKS, chunk_body, None)


@jax.jit
def _sc_call(x, y, tbl4):
    mesh = plsc.VectorSubcoreMesh(
        core_axis_name="c", subcore_axis_name="s",
        num_cores=NUM_CORES, num_subcores=NUM_SUBCORES)
    fn = pl.kernel(
        _sc_body,
        out_type=(
            jax.ShapeDtypeStruct((2, N, C // 2), jnp.float32),
            jax.ShapeDtypeStruct((2, N, C // 2), jnp.float32),
        ),
        mesh=mesh,
        compiler_params=pltpu.CompilerParams(needs_layout_passes=False),
        scratch_types=[
            pltpu.VMEM((GRID * GRID * W_PER_TILE,), jnp.int32),
            pltpu.VMEM((P,), jnp.float32),
            pltpu.VMEM((P,), jnp.float32),
            pltpu.VMEM((P, 2 * W_PER_TILE), jnp.float32),
            pltpu.SemaphoreType.DMA,
        ],
    )
    return fn(x, y, tbl4)


def kernel(xy, gamma, beta, layer_idx):
    tab = jnp.concatenate([gamma[layer_idx], beta[layer_idx]], axis=0)
    tabT = tab.reshape(2 * C, GRID * GRID).T.astype(jnp.bfloat16)
    tbl = lax.bitcast_convert_type(
        tabT.reshape(GRID * GRID, C, 2), jnp.int32)
    tbl4 = tbl.reshape(GRID * GRID, CH_GROUPS, W_PER_TILE).transpose(1, 0, 2)
    tbl4 = tbl4.reshape(CH_GROUPS, GRID * GRID * W_PER_TILE)
    x = xy[:, 0]
    y = xy[:, 1]
    g3, b3 = _sc_call(x, y, tbl4)
    g = g3.transpose(1, 0, 2).reshape(N, C)
    b = b3.transpose(1, 0, 2).reshape(N, C)
    return (g, b)
